# Initial kernel scaffold; baseline (speedup 1.0000x reference)
import jax
import jax.numpy as jnp
from jax.experimental import pallas as pl


def kernel(encoder_inputs, decoder_inputs, src_table, trg_table, pos_table):
    raise NotImplementedError("write your pallas kernel here")


# speedup vs baseline: 1.1665x; 1.1665x over previous
import functools

import jax
import jax.numpy as jnp
from jax import lax
from jax.experimental import pallas as pl
from jax.experimental.pallas import tpu as pltpu
from jax.experimental.pallas import tpu_sc as plsc

PAD = 0
NC, NS, LANES = 2, 16, 16
NW = NC * NS


@functools.partial(jax.jit, static_argnames=())
def _embed(enc_flat, dec_flat, src_table, trg_table, pos_table):
    R = enc_flat.shape[0]
    V, H = src_table.shape
    L = 2048
    rows_per_w = R // NW
    C = 32
    n_chunks = rows_per_w // C

    mesh = plsc.VectorSubcoreMesh(core_axis_name="c", subcore_axis_name="s")

    @functools.partial(
        pl.kernel,
        out_type=(
            jax.ShapeDtypeStruct((R, H), jnp.float32),
            jax.ShapeDtypeStruct((R, H), jnp.float32),
        ),
        mesh=mesh,
        scratch_types=[
            pltpu.VMEM((C,), jnp.int32),
            pltpu.VMEM((C,), jnp.int32),
            pltpu.VMEM((C, H), jnp.float32),
            pltpu.VMEM((C, H), jnp.float32),
            pltpu.SemaphoreType.DMA,
            pltpu.SemaphoreType.DMA,
        ],
    )
    def body(enc_hbm, dec_hbm, src_hbm, trg_hbm, pos_hbm,
             enc_out, dec_out, tok_v, pid_v, wrows, prows, sem_w, sem_p):
        wid = lax.axis_index("s") * NC + lax.axis_index("c")
        base_w = wid * rows_per_w

        def do_chunk(tokens_hbm, table_hbm, out_hbm, ci):
            base = base_w + ci * C
            l0 = lax.rem(base, L)
            pltpu.sync_copy(tokens_hbm.at[pl.ds(base, C)], tok_v)
            for j in range(C // LANES):
                t = tok_v[pl.ds(j * LANES, LANES)]
                pos = lax.iota(jnp.int32, LANES) + (l0 + j * LANES + 1)
                pid_v[pl.ds(j * LANES, LANES)] = jnp.where(t == PAD, 0, pos)
            cp_w = pltpu.async_copy(table_hbm.at[tok_v], wrows, sem_w)
            cp_p = pltpu.async_copy(pos_hbm.at[pid_v], prows, sem_p)
            cp_w.wait()
            cp_p.wait()

            def add_row(r, _):
                for j in range(H // LANES):
                    sl = pl.ds(j * LANES, LANES)
                    wrows[r, sl] = wrows[r, sl] + prows[r, sl]
                return 0

            lax.fori_loop(0, C, add_row, 0)
            pltpu.sync_copy(wrows, out_hbm.at[pl.ds(base, C)])

        for ci in range(n_chunks):
            do_chunk(enc_hbm, src_hbm, enc_out, ci)
            do_chunk(dec_hbm, trg_hbm, dec_out, ci)

    return body(enc_flat, dec_flat, src_table, trg_table, pos_table)


def kernel(encoder_inputs, decoder_inputs, src_table, trg_table, pos_table):
    B, L = encoder_inputs.shape
    H = src_table.shape[1]
    enc_flat = encoder_inputs.reshape(-1).astype(jnp.int32)
    dec_flat = decoder_inputs.reshape(-1).astype(jnp.int32)
    enc_out, dec_out = _embed(enc_flat, dec_flat, src_table, trg_table,
                              pos_table)
    return enc_out.reshape(B, L, H), dec_out.reshape(B, L, H)


# speedup vs baseline: 1.6151x; 1.3845x over previous
import functools

import jax
import jax.numpy as jnp
from jax import lax
from jax.experimental import pallas as pl
from jax.experimental.pallas import tpu as pltpu
from jax.experimental.pallas import tpu_sc as plsc

PAD = 0
NC, NS, LANES = 2, 16, 16
NW = NC * NS


@jax.jit
def _embed(enc_flat, dec_flat, src_table, trg_table, pos_table):
    R = enc_flat.shape[0]
    V, H = src_table.shape
    L = 2048
    rows_per_w = R // NW
    C = 16
    n_chunks = rows_per_w // C

    mesh = plsc.VectorSubcoreMesh(core_axis_name="c", subcore_axis_name="s")

    scratch = []
    for _ in range(2):
        scratch += [
            pltpu.VMEM((C,), jnp.int32),
            pltpu.VMEM((C,), jnp.int32),
            pltpu.VMEM((C, H), jnp.float32),
            pltpu.VMEM((C, H), jnp.float32),
            pltpu.SemaphoreType.DMA,
            pltpu.SemaphoreType.DMA,
            pltpu.SemaphoreType.DMA,
        ]

    @functools.partial(
        pl.kernel,
        out_type=(
            jax.ShapeDtypeStruct((R, H), jnp.float32),
            jax.ShapeDtypeStruct((R, H), jnp.float32),
        ),
        mesh=mesh,
        scratch_types=scratch,
    )
    def body(enc_hbm, dec_hbm, src_hbm, trg_hbm, pos_hbm,
             enc_out, dec_out, *bufs):
        sets = [bufs[i * 7:(i + 1) * 7] for i in range(2)]
        wid = lax.axis_index("s") * NC + lax.axis_index("c")
        base_w = wid * rows_per_w

        chunks = []
        for ci in range(n_chunks):
            chunks.append((enc_hbm, src_hbm, enc_out, ci, 0))
            chunks.append((dec_hbm, trg_hbm, dec_out, ci, 1))

        def issue(k):
            tokens_hbm, table_hbm, _, ci, s = chunks[k]
            tok_v, pid_v, wrows, prows, sem_w, sem_p, _ = sets[s]
            base = base_w + ci * C
            l0 = lax.rem(base, L)
            pltpu.sync_copy(tokens_hbm.at[pl.ds(base, C)], tok_v)
            t = tok_v[...]
            pos = lax.iota(jnp.int32, LANES) + (l0 + 1)
            pid_v[...] = jnp.where(t == PAD, 0, pos)
            cw = pltpu.async_copy(table_hbm.at[tok_v], wrows, sem_w)
            cp = pltpu.async_copy(pos_hbm.at[pid_v], prows, sem_p)
            return cw, cp

        def finish(k, cw, cp):
            _, _, out_hbm, ci, s = chunks[k]
            _, _, wrows, prows, _, _, sem_o = sets[s]
            base = base_w + ci * C
            cw.wait()
            cp.wait()

            def add_row(r, _):
                for j in range(H // LANES):
                    sl = pl.ds(j * LANES, LANES)
                    plsc.addupdate(wrows.at[r, sl], prows[r, sl])
                return 0

            lax.fori_loop(0, C, add_row, 0)
            return pltpu.async_copy(wrows, out_hbm.at[pl.ds(base, C)], sem_o)

        n = len(chunks)
        inflight = {0: issue(0), 1: issue(1)}
        pending_out = {}
        for k in range(n):
            cw, cp = inflight.pop(k)
            out_cp = finish(k, cw, cp)
            s = chunks[k][4]
            pending_out[s] = out_cp
            if k + 2 < n:
                pending_out.pop(s).wait()
                inflight[k + 2] = issue(k + 2)
        for s in sorted(pending_out):
            pending_out[s].wait()

    return body(enc_flat, dec_flat, src_table, trg_table, pos_table)


def kernel(encoder_inputs, decoder_inputs, src_table, trg_table, pos_table):
    B, L = encoder_inputs.shape
    H = src_table.shape[1]
    enc_flat = encoder_inputs.reshape(-1).astype(jnp.int32)
    dec_flat = decoder_inputs.reshape(-1).astype(jnp.int32)
    enc_out, dec_out = _embed(enc_flat, dec_flat, src_table, trg_table,
                              pos_table)
    return enc_out.reshape(B, L, H), dec_out.reshape(B, L, H)
